# baseline (speedup 1.0000x reference)
import functools

import jax
import jax.numpy as jnp
from jax import lax
from jax.experimental import pallas as pl
from jax.experimental.pallas import tpu as pltpu
from jax.experimental.pallas import tpu_sc as plsc

def _z():
    return jnp.int32(0)


def _fori(n, body, init):
    if isinstance(init, int):
        init = jnp.int32(init)
    return lax.fori_loop(jnp.int32(0), jnp.int32(n), body, init)


N_PAD = 10240
CHUNK = 128
NW = 32
NBUF = 2
NSTAGE = 4
D = 128


def _sc_degree(dst_flat):
    nedge = dst_flat.shape[1]
    mesh = plsc.VectorSubcoreMesh(core_axis_name="c", subcore_axis_name="s")

    @functools.partial(
        pl.kernel,
        out_type=jax.ShapeDtypeStruct((NW, N_PAD), jnp.float32),
        mesh=mesh,
        scratch_types=[
            pltpu.VMEM((nedge,), jnp.int32),
            pltpu.VMEM((N_PAD,), jnp.float32),
        ],
        compiler_params=pltpu.CompilerParams(needs_layout_passes=False),
    )
    def deg_kernel(dst_hbm, out_hbm, idx_v, deg_v):
        c = lax.axis_index("c")
        s = lax.axis_index("s")
        w = c * 16 + s
        pltpu.sync_copy(dst_hbm.at[w], idx_v)

        def zero_body(i, carry):
            deg_v[pl.ds(i * 16, 16)] = jnp.zeros((16,), jnp.float32)
            return carry
        _fori(N_PAD // 16, zero_body, 0)

        ones = jnp.ones((16,), jnp.float32)

        def acc_body(i, carry):
            idx = idx_v[pl.ds(i * 16, 16)]
            plsc.addupdate_scatter(deg_v, [idx], ones)
            return carry
        _fori(nedge // 16, acc_body, 0)

        pltpu.sync_copy(deg_v, out_hbm.at[w])

    return deg_kernel(dst_flat)


def _sc_aggregate(hp, src4, dst4, kq0, kq1):
    KQM = src4.shape[2]
    mesh = plsc.VectorSubcoreMesh(core_axis_name="c", subcore_axis_name="s")

    @functools.partial(
        pl.kernel,
        out_type=jax.ShapeDtypeStruct((2, N_PAD, D), jnp.float32),
        mesh=mesh,
        scratch_types=[
            pltpu.VMEM((KQM, CHUNK), jnp.int32),
            pltpu.VMEM((KQM, CHUNK), jnp.int32),
            pltpu.VMEM((NBUF, CHUNK, D), jnp.float32),
            pltpu.VMEM_SHARED((N_PAD, D), jnp.float32),
        ] + [pltpu.SemaphoreType.DMA] * NBUF,
    )
    def agg_kernel(hp_hbm, src_hbm, dst_hbm, out_hbm, sidx, didx, rows, acc, *sems):
        c = lax.axis_index("c")
        s = lax.axis_index("s")
        w = c * 16 + s
        rps = N_PAD // 16
        base = s * rps
        pltpu.sync_copy(hp_hbm.at[pl.ds(base, rps)], acc.at[pl.ds(base, rps)])
        plsc.subcore_barrier()

        nch = jnp.where(c == 0, jnp.int32(kq0), jnp.int32(kq1))
        for st in range(NSTAGE):
            stt = jnp.int32(st)
            pltpu.sync_copy(src_hbm.at[w, stt], sidx)
            pltpu.sync_copy(dst_hbm.at[w, stt], didx)
            for b in range(NBUF):
                bb = jnp.int32(b)
                pltpu.async_copy(hp_hbm.at[sidx.at[bb]], rows.at[bb], sems[b])

            def body(t, carry):
                for b in range(NBUF):
                    bb = jnp.int32(b)
                    j = t * NBUF + bb
                    pltpu.make_async_copy(hp_hbm.at[sidx.at[j]], rows.at[bb],
                                          sems[b]).wait()
                    pltpu.sync_copy(rows.at[bb], acc.at[didx.at[j]], add=True)
                    nj = j + NBUF

                    @pl.when(nj < nch)
                    def _():
                        pltpu.async_copy(hp_hbm.at[sidx.at[nj]], rows.at[bb],
                                         sems[b])
                return carry
            lax.fori_loop(jnp.int32(0), nch // jnp.int32(NBUF), body,
                          jnp.int32(0))

        plsc.subcore_barrier()
        pltpu.sync_copy(acc.at[pl.ds(base, rps)], out_hbm.at[c, pl.ds(base, rps)])

    return agg_kernel(hp, src4, dst4)


def _dis(degp):
    def body(d_ref, o_ref):
        total = jnp.sum(d_ref[...], axis=0) + 1.0
        o_ref[...] = lax.rsqrt(total)[:, None]
    return pl.pallas_call(
        body,
        out_shape=jax.ShapeDtypeStruct((N_PAD, 1), jnp.float32),
    )(degp)


def _rowmm(A, W, A2=None, A3=None, dis=None, pre_dis=False, b_pre=None,
           relu=False, b_post=None, post_dis=False, sigmoid=False):
    BLK = 256
    n, d_in = A.shape
    d_out = W.shape[1]
    grid = (n // BLK,)

    operands = [A, W]
    specs = [pl.BlockSpec((BLK, d_in), lambda i: (i, _z())),
             pl.BlockSpec((d_in, d_out), lambda i: (_z(), _z()))]
    if A2 is not None:
        operands.append(A2)
        specs.append(pl.BlockSpec((BLK, d_in), lambda i: (i, _z())))
    if A3 is not None:
        operands.append(A3)
        specs.append(pl.BlockSpec((BLK, d_in), lambda i: (i, _z())))
    if dis is not None:
        operands.append(dis)
        specs.append(pl.BlockSpec((BLK, 1), lambda i: (i, _z())))
    if b_pre is not None:
        operands.append(b_pre)
        specs.append(pl.BlockSpec((1, d_in), lambda i: (_z(), _z())))
    if b_post is not None:
        operands.append(b_post)
        specs.append(pl.BlockSpec((1, d_out), lambda i: (_z(), _z())))

    def body(*refs):
        it = iter(refs)
        a_ref = next(it)
        w_ref = next(it)
        a2_ref = next(it) if A2 is not None else None
        a3_ref = next(it) if A3 is not None else None
        d_ref = next(it) if dis is not None else None
        bpre_ref = next(it) if b_pre is not None else None
        bpost_ref = next(it) if b_post is not None else None
        o_ref = next(it)

        M = a_ref[...]
        if a2_ref is not None:
            M = M + a2_ref[...]
        if a3_ref is not None:
            M = M - a3_ref[...]
        if pre_dis:
            M = M * d_ref[...]
        if bpre_ref is not None:
            M = M + bpre_ref[...]
        if relu:
            M = jnp.maximum(M, 0.0)
        out = jnp.dot(M, w_ref[...], preferred_element_type=jnp.float32)
        if bpost_ref is not None:
            out = out + bpost_ref[...]
        if post_dis:
            out = out * d_ref[...]
        if sigmoid:
            out = jax.nn.sigmoid(out)
        o_ref[...] = out

    return pl.pallas_call(
        body,
        grid=grid,
        in_specs=specs,
        out_specs=pl.BlockSpec((BLK, d_out), lambda i: (i, _z())),
        out_shape=jax.ShapeDtypeStruct((n, d_out), jnp.float32),
    )(*operands)


def _lstm2(P0, Whh0T, Wcat1, bi1, steps):
    n, g4 = P0.shape
    H = g4 // 4

    def cell(g, c):
        i = jax.nn.sigmoid(g[:, 0 * H:1 * H])
        f = jax.nn.sigmoid(g[:, 1 * H:2 * H])
        gg = jnp.tanh(g[:, 2 * H:3 * H])
        o = jax.nn.sigmoid(g[:, 3 * H:4 * H])
        c2 = f * c + i * gg
        return o * jnp.tanh(c2), c2

    U = 8
    nblk = (steps + U) // U

    def body(p_ref, w0_ref, w1_ref, b1_ref, y_ref):
        w0 = w0_ref[...]
        w1 = w1_ref[...]
        b1v = b1_ref[...]

        def blk(tb, carry):
            h0, c0, h1, c1 = carry
            base = tb * U
            pblk = p_ref[pl.ds(base, U), :]
            outs = []
            for k in range(U):
                g0 = pblk[k:k + 1, :] + jnp.dot(
                    h0, w0, preferred_element_type=jnp.float32)
                h0n, c0n = cell(g0, c0)
                g1 = jnp.dot(jnp.concatenate([h0, h1], axis=1), w1,
                             preferred_element_type=jnp.float32) + b1v
                h1n, c1n = cell(g1, c1)
                if k == 0:
                    valid = base > 0
                    h1n = jnp.where(valid, h1n, 0.0)
                    c1n = jnp.where(valid, c1n, 0.0)
                outs.append(h1n)
                h0, c0, h1, c1 = h0n, c0n, h1n, c1n
            y_ref[pl.ds(base, U), :] = jnp.concatenate(outs, axis=0)
            return (h0, c0, h1, c1)

        z = jnp.zeros((1, H), jnp.float32)
        _fori(nblk, blk, (z, z, z, z))

    return pl.pallas_call(
        body,
        out_shape=jax.ShapeDtypeStruct((nblk * U, H), jnp.float32),
    )(P0, Whh0T, Wcat1, bi1)


def kernel(x, edge_index, W1, b1, W2, b2, W3, b3, W4, b4, W5, b5,
           Wih0, Whh0, bih0, bhh0, Wih1, Whh1, bih1, bhh1, Wfc, bfc):
    n = x.shape[0]
    src = edge_index[0].astype(jnp.int32)
    dst = edge_index[1].astype(jnp.int32)
    E = src.shape[0]
    kq0, kq1 = 36, 4
    cap0 = 16 * NSTAGE * kq0 * CHUNK
    cap1 = 16 * NSTAGE * kq1 * CHUNK
    padn = cap0 + cap1 - E
    src_all = jnp.concatenate([src, jnp.zeros((padn,), jnp.int32)])
    dst_all = jnp.concatenate([dst, jnp.full((padn,), n, jnp.int32)])
    src1 = src_all[:cap1].reshape(16, NSTAGE, kq1, CHUNK)
    dst1 = dst_all[:cap1].reshape(16, NSTAGE, kq1, CHUNK)
    src0 = src_all[cap1:].reshape(16, NSTAGE, kq0, CHUNK)
    dst0 = dst_all[cap1:].reshape(16, NSTAGE, kq0, CHUNK)
    KQM = -(-max(kq0, kq1) // 8) * 8
    src0p = jnp.pad(src0, ((0, 0), (0, 0), (0, KQM - kq0), (0, 0)))
    dst0p = jnp.pad(dst0, ((0, 0), (0, 0), (0, KQM - kq0), (0, 0)),
                    constant_values=n)
    src1p = jnp.pad(src1, ((0, 0), (0, 0), (0, KQM - kq1), (0, 0)))
    dst1p = jnp.pad(dst1, ((0, 0), (0, 0), (0, KQM - kq1), (0, 0)),
                    constant_values=n)
    src_p = jnp.concatenate([src0p, src1p], axis=0)
    dst_p = jnp.concatenate([dst0p, dst1p], axis=0)
    dst_flat = dst_p.reshape(NW, NSTAGE * KQM * CHUNK)
    xp = jnp.pad(x.astype(jnp.float32), ((0, N_PAD - n), (0, 0)))

    b1r = b1.reshape(1, -1)
    bi0 = (bih0 + bhh0).reshape(1, -1)
    bi1 = (bih1 + bhh1).reshape(1, -1)
    Wih0T, Whh0T = Wih0.T, Whh0.T
    Wih1T, Whh1T = Wih1.T, Whh1.T

    degp = _sc_degree(dst_flat)
    dis = _dis(degp)

    hp = _rowmm(xp, W1, dis=dis, post_dis=True)
    p = _sc_aggregate(hp, src_p, dst_p, kq0, kq1)
    P0 = _rowmm(p[0], Wih0T, A2=p[1], A3=hp, dis=dis, pre_dis=True,
                b_pre=b1r, relu=True, b_post=bi0)
    Wcat1 = jnp.concatenate([Wih1T, Whh1T], axis=0)
    Ys = _lstm2(P0, Whh0T, Wcat1, bi1, n)
    Y1 = jnp.pad(Ys[1:], ((0, N_PAD - (Ys.shape[0] - 1)), (0, 0)))

    hp = _rowmm(Y1, W2, dis=dis, post_dis=True)
    for b_k, W_next in ((b2, W3), (b3, W4), (b4, W5)):
        p = _sc_aggregate(hp, src_p, dst_p, kq0, kq1)
        hp = _rowmm(p[0], W_next, A2=p[1], A3=hp, dis=dis, pre_dis=True,
                    b_pre=b_k.reshape(1, -1), relu=True, post_dis=True)
    p = _sc_aggregate(hp, src_p, dst_p, kq0, kq1)
    out = _rowmm(p[0], Wfc, A2=p[1], A3=hp, dis=dis, pre_dis=True,
                 b_pre=b5.reshape(1, -1), relu=True,
                 b_post=bfc.reshape(1, -1), sigmoid=True)
    return out[:n]


# speedup vs baseline: 1.0873x; 1.0873x over previous
import functools

import jax
import jax.numpy as jnp
from jax import lax
from jax.experimental import pallas as pl
from jax.experimental.pallas import tpu as pltpu
from jax.experimental.pallas import tpu_sc as plsc

def _z():
    return jnp.int32(0)


def _fori(n, body, init):
    if isinstance(init, int):
        init = jnp.int32(init)
    return lax.fori_loop(jnp.int32(0), jnp.int32(n), body, init)


N_PAD = 10240
CHUNK = 128
NW = 32
NBUF = 2
NSTAGE = 4
D = 128


def _sc_degree(dst_flat):
    nedge = dst_flat.shape[1]
    mesh = plsc.VectorSubcoreMesh(core_axis_name="c", subcore_axis_name="s")

    @functools.partial(
        pl.kernel,
        out_type=jax.ShapeDtypeStruct((NW, N_PAD), jnp.float32),
        mesh=mesh,
        scratch_types=[
            pltpu.VMEM((nedge,), jnp.int32),
            pltpu.VMEM((N_PAD,), jnp.float32),
        ],
        compiler_params=pltpu.CompilerParams(needs_layout_passes=False),
    )
    def deg_kernel(dst_hbm, out_hbm, idx_v, deg_v):
        c = lax.axis_index("c")
        s = lax.axis_index("s")
        w = c * 16 + s
        pltpu.sync_copy(dst_hbm.at[w], idx_v)

        def zero_body(i, carry):
            deg_v[pl.ds(i * 16, 16)] = jnp.zeros((16,), jnp.float32)
            return carry
        _fori(N_PAD // 16, zero_body, 0)

        ones = jnp.ones((16,), jnp.float32)

        def acc_body(i, carry):
            idx = idx_v[pl.ds(i * 16, 16)]
            plsc.addupdate_scatter(deg_v, [idx], ones)
            return carry
        _fori(nedge // 16, acc_body, 0)

        pltpu.sync_copy(deg_v, out_hbm.at[w])

    return deg_kernel(dst_flat)


def _sc_aggregate(hp, src4, dst4, kq0, kq1):
    KQM = src4.shape[2]
    mesh = plsc.VectorSubcoreMesh(core_axis_name="c", subcore_axis_name="s")

    @functools.partial(
        pl.kernel,
        out_type=jax.ShapeDtypeStruct((2, N_PAD, D), jnp.float32),
        mesh=mesh,
        scratch_types=[
            pltpu.VMEM((KQM, CHUNK), jnp.int32),
            pltpu.VMEM((KQM, CHUNK), jnp.int32),
            pltpu.VMEM((NBUF, CHUNK, D), jnp.float32),
            pltpu.VMEM_SHARED((N_PAD, D), jnp.float32),
        ] + [pltpu.SemaphoreType.DMA] * NBUF,
    )
    def agg_kernel(hp_hbm, src_hbm, dst_hbm, out_hbm, sidx, didx, rows, acc, *sems):
        c = lax.axis_index("c")
        s = lax.axis_index("s")
        w = c * 16 + s
        rps = N_PAD // 16
        base = s * rps
        pltpu.sync_copy(hp_hbm.at[pl.ds(base, rps)], acc.at[pl.ds(base, rps)])
        plsc.subcore_barrier()

        nch = jnp.where(c == 0, jnp.int32(kq0), jnp.int32(kq1))
        for st in range(NSTAGE):
            stt = jnp.int32(st)
            pltpu.sync_copy(src_hbm.at[w, stt], sidx)
            pltpu.sync_copy(dst_hbm.at[w, stt], didx)
            for b in range(NBUF):
                bb = jnp.int32(b)
                pltpu.async_copy(hp_hbm.at[sidx.at[bb]], rows.at[bb], sems[b])

            def body(t, carry):
                for b in range(NBUF):
                    bb = jnp.int32(b)
                    j = t * NBUF + bb
                    pltpu.make_async_copy(hp_hbm.at[sidx.at[j]], rows.at[bb],
                                          sems[b]).wait()
                    pltpu.sync_copy(rows.at[bb], acc.at[didx.at[j]], add=True)
                    nj = j + NBUF

                    @pl.when(nj < nch)
                    def _():
                        pltpu.async_copy(hp_hbm.at[sidx.at[nj]], rows.at[bb],
                                         sems[b])
                return carry
            lax.fori_loop(jnp.int32(0), nch // jnp.int32(NBUF), body,
                          jnp.int32(0))

        plsc.subcore_barrier()
        pltpu.sync_copy(acc.at[pl.ds(base, rps)], out_hbm.at[c, pl.ds(base, rps)])

    return agg_kernel(hp, src4, dst4)


def _dis(degp):
    def body(d_ref, o_ref):
        total = jnp.sum(d_ref[...], axis=0) + 1.0
        o_ref[...] = lax.rsqrt(total)[:, None]
    return pl.pallas_call(
        body,
        out_shape=jax.ShapeDtypeStruct((N_PAD, 1), jnp.float32),
    )(degp)


def _rowmm(A, W, A2=None, A3=None, dis=None, pre_dis=False, b_pre=None,
           relu=False, b_post=None, post_dis=False, sigmoid=False):
    BLK = 256
    n, d_in = A.shape
    d_out = W.shape[1]
    grid = (n // BLK,)

    operands = [A, W]
    specs = [pl.BlockSpec((BLK, d_in), lambda i: (i, _z())),
             pl.BlockSpec((d_in, d_out), lambda i: (_z(), _z()))]
    if A2 is not None:
        operands.append(A2)
        specs.append(pl.BlockSpec((BLK, d_in), lambda i: (i, _z())))
    if A3 is not None:
        operands.append(A3)
        specs.append(pl.BlockSpec((BLK, d_in), lambda i: (i, _z())))
    if dis is not None:
        operands.append(dis)
        specs.append(pl.BlockSpec((BLK, 1), lambda i: (i, _z())))
    if b_pre is not None:
        operands.append(b_pre)
        specs.append(pl.BlockSpec((1, d_in), lambda i: (_z(), _z())))
    if b_post is not None:
        operands.append(b_post)
        specs.append(pl.BlockSpec((1, d_out), lambda i: (_z(), _z())))

    def body(*refs):
        it = iter(refs)
        a_ref = next(it)
        w_ref = next(it)
        a2_ref = next(it) if A2 is not None else None
        a3_ref = next(it) if A3 is not None else None
        d_ref = next(it) if dis is not None else None
        bpre_ref = next(it) if b_pre is not None else None
        bpost_ref = next(it) if b_post is not None else None
        o_ref = next(it)

        M = a_ref[...]
        if a2_ref is not None:
            M = M + a2_ref[...]
        if a3_ref is not None:
            M = M - a3_ref[...]
        if pre_dis:
            M = M * d_ref[...]
        if bpre_ref is not None:
            M = M + bpre_ref[...]
        if relu:
            M = jnp.maximum(M, 0.0)
        out = jnp.dot(M, w_ref[...], preferred_element_type=jnp.float32)
        if bpost_ref is not None:
            out = out + bpost_ref[...]
        if post_dis:
            out = out * d_ref[...]
        if sigmoid:
            out = jax.nn.sigmoid(out)
        o_ref[...] = out

    return pl.pallas_call(
        body,
        grid=grid,
        in_specs=specs,
        out_specs=pl.BlockSpec((BLK, d_out), lambda i: (i, _z())),
        out_shape=jax.ShapeDtypeStruct((n, d_out), jnp.float32),
    )(*operands)


def _lstm2(P0, Whh0T, Wcat1, bi1, steps):
    n, g4 = P0.shape
    H = g4 // 4

    def cell(g, c):
        i = jax.nn.sigmoid(g[:, 0 * H:1 * H])
        f = jax.nn.sigmoid(g[:, 1 * H:2 * H])
        gg = jnp.tanh(g[:, 2 * H:3 * H])
        o = jax.nn.sigmoid(g[:, 3 * H:4 * H])
        c2 = f * c + i * gg
        return o * jnp.tanh(c2), c2

    U = 8
    nblk = (steps + U) // U

    def body(p_ref, w0_ref, w1_ref, b1_ref, y_ref):
        w0 = w0_ref[...]
        w1 = w1_ref[...]
        b1v = b1_ref[...]

        def blk(tb, carry):
            h0, c0, h1, c1 = carry
            base = tb * U
            pblk = p_ref[pl.ds(base, U), :]
            outs = []
            for k in range(U):
                g0 = pblk[k:k + 1, :] + jnp.dot(
                    h0, w0, preferred_element_type=jnp.float32)
                h0n, c0n = cell(g0, c0)
                g1 = jnp.dot(jnp.concatenate([h0, h1], axis=1), w1,
                             preferred_element_type=jnp.float32) + b1v
                h1n, c1n = cell(g1, c1)
                if k == 0:
                    valid = base > 0
                    h1n = jnp.where(valid, h1n, 0.0)
                    c1n = jnp.where(valid, c1n, 0.0)
                outs.append(h1n)
                h0, c0, h1, c1 = h0n, c0n, h1n, c1n
            y_ref[pl.ds(base, U), :] = jnp.concatenate(outs, axis=0)
            return (h0, c0, h1, c1)

        z = jnp.zeros((1, H), jnp.float32)
        _fori(nblk, blk, (z, z, z, z))

    return pl.pallas_call(
        body,
        out_shape=jax.ShapeDtypeStruct((nblk * U, H), jnp.float32),
    )(P0, Whh0T, Wcat1, bi1)


def kernel(x, edge_index, W1, b1, W2, b2, W3, b3, W4, b4, W5, b5,
           Wih0, Whh0, bih0, bhh0, Wih1, Whh1, bih1, bhh1, Wfc, bfc):
    n = x.shape[0]
    src = edge_index[0].astype(jnp.int32)
    dst = edge_index[1].astype(jnp.int32)
    E = src.shape[0]
    kq0, kq1 = 28, 12
    cap0 = 16 * NSTAGE * kq0 * CHUNK
    cap1 = 16 * NSTAGE * kq1 * CHUNK
    padn = cap0 + cap1 - E
    src_all = jnp.concatenate([src, jnp.zeros((padn,), jnp.int32)])
    dst_all = jnp.concatenate([dst, jnp.full((padn,), n, jnp.int32)])
    src1 = src_all[:cap1].reshape(16, NSTAGE, kq1, CHUNK)
    dst1 = dst_all[:cap1].reshape(16, NSTAGE, kq1, CHUNK)
    src0 = src_all[cap1:].reshape(16, NSTAGE, kq0, CHUNK)
    dst0 = dst_all[cap1:].reshape(16, NSTAGE, kq0, CHUNK)
    KQM = -(-max(kq0, kq1) // 8) * 8
    src0p = jnp.pad(src0, ((0, 0), (0, 0), (0, KQM - kq0), (0, 0)))
    dst0p = jnp.pad(dst0, ((0, 0), (0, 0), (0, KQM - kq0), (0, 0)),
                    constant_values=n)
    src1p = jnp.pad(src1, ((0, 0), (0, 0), (0, KQM - kq1), (0, 0)))
    dst1p = jnp.pad(dst1, ((0, 0), (0, 0), (0, KQM - kq1), (0, 0)),
                    constant_values=n)
    src_p = jnp.concatenate([src0p, src1p], axis=0)
    dst_p = jnp.concatenate([dst0p, dst1p], axis=0)
    dst_flat = dst_p.reshape(NW, NSTAGE * KQM * CHUNK)
    xp = jnp.pad(x.astype(jnp.float32), ((0, N_PAD - n), (0, 0)))

    b1r = b1.reshape(1, -1)
    bi0 = (bih0 + bhh0).reshape(1, -1)
    bi1 = (bih1 + bhh1).reshape(1, -1)
    Wih0T, Whh0T = Wih0.T, Whh0.T
    Wih1T, Whh1T = Wih1.T, Whh1.T

    degp = _sc_degree(dst_flat)
    dis = _dis(degp)

    hp = _rowmm(xp, W1, dis=dis, post_dis=True)
    p = _sc_aggregate(hp, src_p, dst_p, kq0, kq1)
    P0 = _rowmm(p[0], Wih0T, A2=p[1], A3=hp, dis=dis, pre_dis=True,
                b_pre=b1r, relu=True, b_post=bi0)
    Wcat1 = jnp.concatenate([Wih1T, Whh1T], axis=0)
    Ys = _lstm2(P0, Whh0T, Wcat1, bi1, n)
    Y1 = jnp.pad(Ys[1:], ((0, N_PAD - (Ys.shape[0] - 1)), (0, 0)))

    hp = _rowmm(Y1, W2, dis=dis, post_dis=True)
    for b_k, W_next in ((b2, W3), (b3, W4), (b4, W5)):
        p = _sc_aggregate(hp, src_p, dst_p, kq0, kq1)
        hp = _rowmm(p[0], W_next, A2=p[1], A3=hp, dis=dis, pre_dis=True,
                    b_pre=b_k.reshape(1, -1), relu=True, post_dis=True)
    p = _sc_aggregate(hp, src_p, dst_p, kq0, kq1)
    out = _rowmm(p[0], Wfc, A2=p[1], A3=hp, dis=dis, pre_dis=True,
                 b_pre=b5.reshape(1, -1), relu=True,
                 b_post=bfc.reshape(1, -1), sigmoid=True)
    return out[:n]


# speedup vs baseline: 1.1207x; 1.0308x over previous
import functools

import jax
import jax.numpy as jnp
from jax import lax
from jax.experimental import pallas as pl
from jax.experimental.pallas import tpu as pltpu
from jax.experimental.pallas import tpu_sc as plsc

def _z():
    return jnp.int32(0)


def _fori(n, body, init):
    if isinstance(init, int):
        init = jnp.int32(init)
    return lax.fori_loop(jnp.int32(0), jnp.int32(n), body, init)


N_PAD = 10240
CHUNK = 128
NW = 32
NBUF = 2
NSTAGE = 4
D = 128


def _sc_degree(dst_flat):
    nedge = dst_flat.shape[1]
    mesh = plsc.VectorSubcoreMesh(core_axis_name="c", subcore_axis_name="s")

    @functools.partial(
        pl.kernel,
        out_type=jax.ShapeDtypeStruct((NW, N_PAD), jnp.float32),
        mesh=mesh,
        scratch_types=[
            pltpu.VMEM((nedge,), jnp.int32),
            pltpu.VMEM((N_PAD,), jnp.float32),
        ],
        compiler_params=pltpu.CompilerParams(needs_layout_passes=False),
    )
    def deg_kernel(dst_hbm, out_hbm, idx_v, deg_v):
        c = lax.axis_index("c")
        s = lax.axis_index("s")
        w = c * 16 + s
        pltpu.sync_copy(dst_hbm.at[w], idx_v)

        def zero_body(i, carry):
            deg_v[pl.ds(i * 16, 16)] = jnp.zeros((16,), jnp.float32)
            return carry
        _fori(N_PAD // 16, zero_body, 0)

        ones = jnp.ones((16,), jnp.float32)

        def acc_body(i, carry):
            idx = idx_v[pl.ds(i * 16, 16)]
            plsc.addupdate_scatter(deg_v, [idx], ones)
            return carry
        _fori(nedge // 16, acc_body, 0)

        pltpu.sync_copy(deg_v, out_hbm.at[w])

    return deg_kernel(dst_flat)


def _sc_aggregate(hp, src4, dst4, kq0, kq1):
    KQM = src4.shape[2]
    mesh = plsc.VectorSubcoreMesh(core_axis_name="c", subcore_axis_name="s")

    @functools.partial(
        pl.kernel,
        out_type=jax.ShapeDtypeStruct((2, N_PAD, D), jnp.float32),
        mesh=mesh,
        scratch_types=[
            pltpu.VMEM((KQM, CHUNK), jnp.int32),
            pltpu.VMEM((KQM, CHUNK), jnp.int32),
            pltpu.VMEM((NBUF, CHUNK, D), jnp.float32),
            pltpu.VMEM_SHARED((N_PAD, D), jnp.float32),
        ] + [pltpu.SemaphoreType.DMA] * NBUF,
    )
    def agg_kernel(hp_hbm, src_hbm, dst_hbm, out_hbm, sidx, didx, rows, acc, *sems):
        c = lax.axis_index("c")
        s = lax.axis_index("s")
        w = c * 16 + s
        rps = N_PAD // 16
        base = s * rps
        pltpu.sync_copy(hp_hbm.at[pl.ds(base, rps)], acc.at[pl.ds(base, rps)])
        plsc.subcore_barrier()

        nch = jnp.where(c == 0, jnp.int32(kq0), jnp.int32(kq1))
        for st in range(NSTAGE):
            stt = jnp.int32(st)
            pltpu.sync_copy(src_hbm.at[w, stt], sidx)
            pltpu.sync_copy(dst_hbm.at[w, stt], didx)
            for b in range(NBUF):
                bb = jnp.int32(b)
                pltpu.async_copy(hp_hbm.at[sidx.at[bb]], rows.at[bb], sems[b])

            def body(t, carry):
                for b in range(NBUF):
                    bb = jnp.int32(b)
                    j = t * NBUF + bb
                    pltpu.make_async_copy(hp_hbm.at[sidx.at[j]], rows.at[bb],
                                          sems[b]).wait()
                    pltpu.sync_copy(rows.at[bb], acc.at[didx.at[j]], add=True)
                    nj = j + NBUF

                    @pl.when(nj < nch)
                    def _():
                        pltpu.async_copy(hp_hbm.at[sidx.at[nj]], rows.at[bb],
                                         sems[b])
                return carry
            lax.fori_loop(jnp.int32(0), nch // jnp.int32(NBUF), body,
                          jnp.int32(0))

        plsc.subcore_barrier()
        pltpu.sync_copy(acc.at[pl.ds(base, rps)], out_hbm.at[c, pl.ds(base, rps)])

    return agg_kernel(hp, src4, dst4)


def _dis(degp):
    def body(d_ref, o_ref):
        total = jnp.sum(d_ref[...], axis=0) + 1.0
        o_ref[...] = lax.rsqrt(total)[:, None]
    return pl.pallas_call(
        body,
        out_shape=jax.ShapeDtypeStruct((N_PAD, 1), jnp.float32),
    )(degp)


def _rowmm(A, W, A2=None, A3=None, dis=None, pre_dis=False, b_pre=None,
           relu=False, b_post=None, post_dis=False, sigmoid=False):
    BLK = 256
    n, d_in = A.shape
    d_out = W.shape[1]
    grid = (n // BLK,)

    operands = [A, W]
    specs = [pl.BlockSpec((BLK, d_in), lambda i: (i, _z())),
             pl.BlockSpec((d_in, d_out), lambda i: (_z(), _z()))]
    if A2 is not None:
        operands.append(A2)
        specs.append(pl.BlockSpec((BLK, d_in), lambda i: (i, _z())))
    if A3 is not None:
        operands.append(A3)
        specs.append(pl.BlockSpec((BLK, d_in), lambda i: (i, _z())))
    if dis is not None:
        operands.append(dis)
        specs.append(pl.BlockSpec((BLK, 1), lambda i: (i, _z())))
    if b_pre is not None:
        operands.append(b_pre)
        specs.append(pl.BlockSpec((1, d_in), lambda i: (_z(), _z())))
    if b_post is not None:
        operands.append(b_post)
        specs.append(pl.BlockSpec((1, d_out), lambda i: (_z(), _z())))

    def body(*refs):
        it = iter(refs)
        a_ref = next(it)
        w_ref = next(it)
        a2_ref = next(it) if A2 is not None else None
        a3_ref = next(it) if A3 is not None else None
        d_ref = next(it) if dis is not None else None
        bpre_ref = next(it) if b_pre is not None else None
        bpost_ref = next(it) if b_post is not None else None
        o_ref = next(it)

        M = a_ref[...]
        if a2_ref is not None:
            M = M + a2_ref[...]
        if a3_ref is not None:
            M = M - a3_ref[...]
        if pre_dis:
            M = M * d_ref[...]
        if bpre_ref is not None:
            M = M + bpre_ref[...]
        if relu:
            M = jnp.maximum(M, 0.0)
        out = jnp.dot(M, w_ref[...], preferred_element_type=jnp.float32)
        if bpost_ref is not None:
            out = out + bpost_ref[...]
        if post_dis:
            out = out * d_ref[...]
        if sigmoid:
            out = jax.nn.sigmoid(out)
        o_ref[...] = out

    return pl.pallas_call(
        body,
        grid=grid,
        in_specs=specs,
        out_specs=pl.BlockSpec((BLK, d_out), lambda i: (i, _z())),
        out_shape=jax.ShapeDtypeStruct((n, d_out), jnp.float32),
    )(*operands)


def _lstm2(P0, Whh0T, Wcat1, bi1, steps):
    n, g4 = P0.shape
    H = g4 // 4

    def cell(g, c):
        i = jax.nn.sigmoid(g[:, 0 * H:1 * H])
        f = jax.nn.sigmoid(g[:, 1 * H:2 * H])
        gg = jnp.tanh(g[:, 2 * H:3 * H])
        o = jax.nn.sigmoid(g[:, 3 * H:4 * H])
        c2 = f * c + i * gg
        return o * jnp.tanh(c2), c2

    U = 8
    nblk = (steps + U) // U

    def body(p_ref, w0_ref, w1_ref, b1_ref, y_ref):
        w0 = w0_ref[...]
        w1 = w1_ref[...]
        b1v = b1_ref[...]

        def blk(tb, carry):
            h0, c0, h1, c1 = carry
            base = tb * U
            pblk = p_ref[pl.ds(base, U), :]
            outs = []
            for k in range(U):
                g0 = pblk[k:k + 1, :] + jnp.dot(
                    h0, w0, preferred_element_type=jnp.float32)
                h0n, c0n = cell(g0, c0)
                g1 = jnp.dot(jnp.concatenate([h0, h1], axis=1), w1,
                             preferred_element_type=jnp.float32) + b1v
                h1n, c1n = cell(g1, c1)
                if k == 0:
                    valid = base > 0
                    h1n = jnp.where(valid, h1n, 0.0)
                    c1n = jnp.where(valid, c1n, 0.0)
                outs.append(h1n)
                h0, c0, h1, c1 = h0n, c0n, h1n, c1n
            y_ref[pl.ds(base, U), :] = jnp.concatenate(outs, axis=0)
            return (h0, c0, h1, c1)

        z = jnp.zeros((1, H), jnp.float32)
        _fori(nblk, blk, (z, z, z, z))

    return pl.pallas_call(
        body,
        out_shape=jax.ShapeDtypeStruct((nblk * U, H), jnp.float32),
    )(P0, Whh0T, Wcat1, bi1)


def kernel(x, edge_index, W1, b1, W2, b2, W3, b3, W4, b4, W5, b5,
           Wih0, Whh0, bih0, bhh0, Wih1, Whh1, bih1, bhh1, Wfc, bfc):
    n = x.shape[0]
    src = edge_index[0].astype(jnp.int32)
    dst = edge_index[1].astype(jnp.int32)
    E = src.shape[0]
    kq0, kq1 = 32, 8
    cap0 = 16 * NSTAGE * kq0 * CHUNK
    cap1 = 16 * NSTAGE * kq1 * CHUNK
    padn = cap0 + cap1 - E
    src_all = jnp.concatenate([src, jnp.zeros((padn,), jnp.int32)])
    dst_all = jnp.concatenate([dst, jnp.full((padn,), n, jnp.int32)])
    src1 = src_all[:cap1].reshape(16, NSTAGE, kq1, CHUNK)
    dst1 = dst_all[:cap1].reshape(16, NSTAGE, kq1, CHUNK)
    src0 = src_all[cap1:].reshape(16, NSTAGE, kq0, CHUNK)
    dst0 = dst_all[cap1:].reshape(16, NSTAGE, kq0, CHUNK)
    KQM = -(-max(kq0, kq1) // 8) * 8
    src0p = jnp.pad(src0, ((0, 0), (0, 0), (0, KQM - kq0), (0, 0)))
    dst0p = jnp.pad(dst0, ((0, 0), (0, 0), (0, KQM - kq0), (0, 0)),
                    constant_values=n)
    src1p = jnp.pad(src1, ((0, 0), (0, 0), (0, KQM - kq1), (0, 0)))
    dst1p = jnp.pad(dst1, ((0, 0), (0, 0), (0, KQM - kq1), (0, 0)),
                    constant_values=n)
    src_p = jnp.concatenate([src0p, src1p], axis=0)
    dst_p = jnp.concatenate([dst0p, dst1p], axis=0)
    dst_flat = dst_p.reshape(NW, NSTAGE * KQM * CHUNK)
    xp = jnp.pad(x.astype(jnp.float32), ((0, N_PAD - n), (0, 0)))

    b1r = b1.reshape(1, -1)
    bi0 = (bih0 + bhh0).reshape(1, -1)
    bi1 = (bih1 + bhh1).reshape(1, -1)
    Wih0T, Whh0T = Wih0.T, Whh0.T
    Wih1T, Whh1T = Wih1.T, Whh1.T

    degp = _sc_degree(dst_flat)
    dis = _dis(degp)

    hp = _rowmm(xp, W1, dis=dis, post_dis=True)
    p = _sc_aggregate(hp, src_p, dst_p, kq0, kq1)
    P0 = _rowmm(p[0], Wih0T, A2=p[1], A3=hp, dis=dis, pre_dis=True,
                b_pre=b1r, relu=True, b_post=bi0)
    Wcat1 = jnp.concatenate([Wih1T, Whh1T], axis=0)
    Ys = _lstm2(P0, Whh0T, Wcat1, bi1, n)
    Y1 = jnp.pad(Ys[1:], ((0, N_PAD - (Ys.shape[0] - 1)), (0, 0)))

    hp = _rowmm(Y1, W2, dis=dis, post_dis=True)
    for b_k, W_next in ((b2, W3), (b3, W4), (b4, W5)):
        p = _sc_aggregate(hp, src_p, dst_p, kq0, kq1)
        hp = _rowmm(p[0], W_next, A2=p[1], A3=hp, dis=dis, pre_dis=True,
                    b_pre=b_k.reshape(1, -1), relu=True, post_dis=True)
    p = _sc_aggregate(hp, src_p, dst_p, kq0, kq1)
    out = _rowmm(p[0], Wfc, A2=p[1], A3=hp, dis=dis, pre_dis=True,
                 b_pre=b5.reshape(1, -1), relu=True,
                 b_post=bfc.reshape(1, -1), sigmoid=True)
    return out[:n]
